# baseline (speedup 1.0000x reference)
import math

import jax
import jax.numpy as jnp
from jax.experimental import pallas as pl
from jax.experimental.pallas import tpu as pltpu

_B, _S, _D = 1, 2048, 768
_H = 12
_HD = _D // _H
_WINDOW = 64
_HALF = _WINDOW // 2

_BQ = 256
_LK = _BQ + _WINDOW
_NBLK = _S // _BQ
_TQ = 64
_TK = 2 * _TQ
_NT = _BQ // _TQ
_SCALE = 1.0 / math.sqrt(_HD)


def _attn_body(x_ref, wq_ref, wk_ref, wv_ref, bq_ref, bk_ref, bv_ref,
               wo_ref, bo_ref, mask_ref, o_ref, k_s, v_s, val_s):
    i = pl.program_id(0)
    qs = pl.multiple_of(i * _BQ, _BQ)
    start = pl.multiple_of(
        jnp.minimum(jnp.maximum(qs - _HALF, 0), _S - _LK), _HALF)

    xb = x_ref[pl.ds(qs, _BQ), :].astype(jnp.bfloat16)
    xs = x_ref[pl.ds(start, _LK), :].astype(jnp.bfloat16)

    q = jnp.dot(xb, wq_ref[...], preferred_element_type=jnp.float32)
    q = q + bq_ref[0, :][None, :]
    k = jnp.dot(xs, wk_ref[...], preferred_element_type=jnp.float32)
    k = k + bk_ref[0, :][None, :]
    v = jnp.dot(xs, wv_ref[...], preferred_element_type=jnp.float32)
    v = v + bv_ref[0, :][None, :]

    qb = (q * _SCALE).astype(jnp.bfloat16)
    k_s[...] = k.astype(jnp.bfloat16)
    v_s[...] = v.astype(jnp.bfloat16)

    for t in range(_NT):
        off = pl.multiple_of(
            jnp.minimum(jnp.maximum(qs - start + t * _TQ - _HALF, 0),
                        _LK - _TK), _HALF)
        i_abs = (qs + t * _TQ) + jax.lax.broadcasted_iota(
            jnp.int32, (_TQ, _TK), 0)
        j_abs = (start + off) + jax.lax.broadcasted_iota(
            jnp.int32, (_TQ, _TK), 1)
        band = (j_abs >= i_abs - _HALF) & (j_abs <= i_abs + _HALF)
        kpad = jnp.transpose(
            mask_ref[pl.ds(pl.multiple_of(start + off, _HALF), _TK), :]
        )
        valid = band & (kpad != 0)
        addmask = jnp.where(valid, 0.0, -1e30).astype(jnp.float32)

        for h in range(_H):
            sl = slice(h * _HD, (h + 1) * _HD)
            qt = qb[t * _TQ:(t + 1) * _TQ, sl]
            kt = k_s[pl.ds(off, _TK), sl]
            logits = jax.lax.dot_general(
                qt, kt, (((1,), (1,)), ((), ())),
                preferred_element_type=jnp.float32)
            e = jnp.exp(logits + addmask)
            s = jnp.sum(e, axis=1, keepdims=True)
            att = (e * (1.0 / s)).astype(jnp.bfloat16)
            vt = v_s[pl.ds(off, _TK), sl]
            val_s[t * _TQ:(t + 1) * _TQ, sl] = jax.lax.dot_general(
                att, vt, (((1,), (0,)), ((), ())),
                preferred_element_type=jnp.float32).astype(jnp.bfloat16)

    values = val_s[...].astype(jnp.float32)
    qpad = mask_ref[pl.ds(qs, _BQ), :]
    values = jnp.where(qpad != 0, values, 0.0)

    out = jnp.dot(values.astype(jnp.bfloat16), wo_ref[...],
                  preferred_element_type=jnp.float32)
    o_ref[...] = out + bo_ref[0, :][None, :]


def kernel(x, padding_mask, Wqkv, bqkv, Wo, bo):
    w = Wqkv.reshape(_H, 3, _HD, _D)
    wq = w[:, 0].reshape(_D, _D).T.astype(jnp.bfloat16)
    wk = w[:, 1].reshape(_D, _D).T.astype(jnp.bfloat16)
    wv = w[:, 2].reshape(_D, _D).T.astype(jnp.bfloat16)
    b3 = bqkv.reshape(_H, 3, _HD)
    bq = b3[:, 0].reshape(1, _D)
    bk = b3[:, 1].reshape(1, _D)
    bv = b3[:, 2].reshape(1, _D)

    wo = Wo.T.astype(jnp.bfloat16)
    bo2 = bo.reshape(1, _D)
    mask2 = padding_mask.reshape(_S, 1).astype(jnp.float32)
    x2 = x.reshape(_S, _D)

    out = pl.pallas_call(
        _attn_body,
        grid=(_NBLK,),
        in_specs=[
            pl.BlockSpec((_S, _D), lambda i: (0, 0)),
            pl.BlockSpec((_D, _D), lambda i: (0, 0)),
            pl.BlockSpec((_D, _D), lambda i: (0, 0)),
            pl.BlockSpec((_D, _D), lambda i: (0, 0)),
            pl.BlockSpec((1, _D), lambda i: (0, 0)),
            pl.BlockSpec((1, _D), lambda i: (0, 0)),
            pl.BlockSpec((1, _D), lambda i: (0, 0)),
            pl.BlockSpec((_D, _D), lambda i: (0, 0)),
            pl.BlockSpec((1, _D), lambda i: (0, 0)),
            pl.BlockSpec((_S, 1), lambda i: (0, 0)),
        ],
        out_specs=pl.BlockSpec((_BQ, _D), lambda i: (i, 0)),
        out_shape=jax.ShapeDtypeStruct((_S, _D), jnp.float32),
        scratch_shapes=[
            pltpu.VMEM((_LK, _D), jnp.bfloat16),
            pltpu.VMEM((_LK, _D), jnp.bfloat16),
            pltpu.VMEM((_BQ, _D), jnp.bfloat16),
        ],
    )(x2, wq, wk, wv, bq, bk, bv, wo, bo2, mask2)

    return out.reshape(_B, _S, _D)


# speedup vs baseline: 1.3807x; 1.3807x over previous
import math

import jax
import jax.numpy as jnp
from jax.experimental import pallas as pl

_B, _S, _D = 1, 2048, 768
_H = 12
_HD = _D // _H
_WINDOW = 64
_HALF = _WINDOW // 2

_BQ = 128
_LK = _BQ + _WINDOW
_NBLK = _S // _BQ
_SCALE = 1.0 / math.sqrt(_HD)
_NEG = -9e15


def _attn_body(x_ref, wq_ref, wk_ref, wv_ref, bq_ref, bk_ref, bv_ref,
               wo_ref, bo_ref, mask_ref, o_ref):
    i = pl.program_id(0)
    qs = pl.multiple_of(i * _BQ, _BQ)
    start = pl.multiple_of(
        jnp.minimum(jnp.maximum(qs - _HALF, 0), _S - _LK), _HALF)

    xb = x_ref[pl.ds(qs, _BQ), :].astype(jnp.bfloat16)
    xs = x_ref[pl.ds(start, _LK), :].astype(jnp.bfloat16)

    q = jnp.dot(xb, wq_ref[...], preferred_element_type=jnp.float32)
    q = q + bq_ref[0, :][None, :]
    k = jnp.dot(xs, wk_ref[...], preferred_element_type=jnp.float32)
    k = k + bk_ref[0, :][None, :]
    v = jnp.dot(xs, wv_ref[...], preferred_element_type=jnp.float32)
    v = v + bv_ref[0, :][None, :]

    i_abs = qs + jax.lax.broadcasted_iota(jnp.int32, (_BQ, _LK), 0)
    j_abs = start + jax.lax.broadcasted_iota(jnp.int32, (_BQ, _LK), 1)
    band = (j_abs >= i_abs - _HALF) & (j_abs <= i_abs + _HALF)
    kpad = jnp.transpose(mask_ref[pl.ds(start, _LK), :])
    valid = band & (kpad != 0)
    addmask = jnp.where(valid, 0.0, -1e30).astype(jnp.float32)

    qb = (q * _SCALE).astype(jnp.bfloat16)
    kb = k.astype(jnp.bfloat16)
    vb = v.astype(jnp.bfloat16)

    vals = []
    for h in range(_H):
        sl = slice(h * _HD, (h + 1) * _HD)
        logits = jax.lax.dot_general(
            qb[:, sl], kb[:, sl],
            (((1,), (1,)), ((), ())),
            preferred_element_type=jnp.float32,
        )
        e = jnp.exp(logits + addmask)
        s = jnp.sum(e, axis=1, keepdims=True)
        att = (e * (1.0 / s)).astype(jnp.bfloat16)
        vals.append(jax.lax.dot_general(
            att, vb[:, sl],
            (((1,), (0,)), ((), ())),
            preferred_element_type=jnp.float32,
        ))

    values = jnp.concatenate(vals, axis=1)
    qpad = mask_ref[pl.ds(qs, _BQ), :]
    values = jnp.where(qpad != 0, values, 0.0)

    out = jnp.dot(values.astype(jnp.bfloat16), wo_ref[...],
                  preferred_element_type=jnp.float32)
    o_ref[...] = out + bo_ref[0, :][None, :]


def kernel(x, padding_mask, Wqkv, bqkv, Wo, bo):
    w = Wqkv.reshape(_H, 3, _HD, _D)
    wq = w[:, 0].reshape(_D, _D).T.astype(jnp.bfloat16)
    wk = w[:, 1].reshape(_D, _D).T.astype(jnp.bfloat16)
    wv = w[:, 2].reshape(_D, _D).T.astype(jnp.bfloat16)
    b3 = bqkv.reshape(_H, 3, _HD)
    bq = b3[:, 0].reshape(1, _D)
    bk = b3[:, 1].reshape(1, _D)
    bv = b3[:, 2].reshape(1, _D)

    wo = Wo.T.astype(jnp.bfloat16)
    bo2 = bo.reshape(1, _D)
    mask2 = padding_mask.reshape(_S, 1).astype(jnp.float32)
    x2 = x.reshape(_S, _D)

    out = pl.pallas_call(
        _attn_body,
        grid=(_NBLK,),
        in_specs=[
            pl.BlockSpec((_S, _D), lambda i: (0, 0)),
            pl.BlockSpec((_D, _D), lambda i: (0, 0)),
            pl.BlockSpec((_D, _D), lambda i: (0, 0)),
            pl.BlockSpec((_D, _D), lambda i: (0, 0)),
            pl.BlockSpec((1, _D), lambda i: (0, 0)),
            pl.BlockSpec((1, _D), lambda i: (0, 0)),
            pl.BlockSpec((1, _D), lambda i: (0, 0)),
            pl.BlockSpec((_D, _D), lambda i: (0, 0)),
            pl.BlockSpec((1, _D), lambda i: (0, 0)),
            pl.BlockSpec((_S, 1), lambda i: (0, 0)),
        ],
        out_specs=pl.BlockSpec((_BQ, _D), lambda i: (i, 0)),
        out_shape=jax.ShapeDtypeStruct((_S, _D), jnp.float32),
    )(x2, wq, wk, wv, bq, bk, bv, wo, bo2, mask2)

    return out.reshape(_B, _S, _D)


# speedup vs baseline: 2.1206x; 1.5360x over previous
import math

import jax
import jax.numpy as jnp
from jax.experimental import pallas as pl

_B, _S, _D = 1, 2048, 768
_H = 12
_HD = _D // _H
_WINDOW = 64
_HALF = _WINDOW // 2

_BQ = 256
_LK = _BQ + _WINDOW
_NBLK = _S // _BQ
_SCALE = 1.0 / math.sqrt(_HD)
_NEG = -9e15


def _attn_body(x_ref, wq_ref, wk_ref, wv_ref, bq_ref, bk_ref, bv_ref,
               wo_ref, bo_ref, mask_ref, o_ref):
    i = pl.program_id(0)
    qs = pl.multiple_of(i * _BQ, _BQ)
    start = pl.multiple_of(
        jnp.minimum(jnp.maximum(qs - _HALF, 0), _S - _LK), _HALF)

    xb = x_ref[pl.ds(qs, _BQ), :].astype(jnp.bfloat16)
    xs = x_ref[pl.ds(start, _LK), :].astype(jnp.bfloat16)

    q = jnp.dot(xb, wq_ref[...], preferred_element_type=jnp.float32)
    q = q + bq_ref[0, :][None, :]
    k = jnp.dot(xs, wk_ref[...], preferred_element_type=jnp.float32)
    k = k + bk_ref[0, :][None, :]
    v = jnp.dot(xs, wv_ref[...], preferred_element_type=jnp.float32)
    v = v + bv_ref[0, :][None, :]

    i_abs = qs + jax.lax.broadcasted_iota(jnp.int32, (_BQ, _LK), 0)
    j_abs = start + jax.lax.broadcasted_iota(jnp.int32, (_BQ, _LK), 1)
    band = (j_abs >= i_abs - _HALF) & (j_abs <= i_abs + _HALF)
    kpad = jnp.transpose(mask_ref[pl.ds(start, _LK), :])
    valid = band & (kpad != 0)
    addmask = jnp.where(valid, 0.0, -1e30).astype(jnp.float32)

    qb = (q * _SCALE).astype(jnp.bfloat16)
    kb = k.astype(jnp.bfloat16)
    vb = v.astype(jnp.bfloat16)

    qpad = mask_ref[pl.ds(qs, _BQ), :]

    vals = []
    for h in range(_H):
        sl = slice(h * _HD, (h + 1) * _HD)
        logits = jax.lax.dot_general(
            qb[:, sl], kb[:, sl],
            (((1,), (1,)), ((), ())),
            preferred_element_type=jnp.float32,
        )
        eb = jnp.exp(logits + addmask).astype(jnp.bfloat16)
        s = jnp.sum(eb, axis=1, keepdims=True,
                    dtype=jnp.float32)
        u = jax.lax.dot_general(
            eb, vb[:, sl],
            (((1,), (0,)), ((), ())),
            preferred_element_type=jnp.float32,
        )
        vals.append((u * (qpad * (1.0 / s))).astype(jnp.bfloat16))

    values = jnp.concatenate(vals, axis=1)

    out = jnp.dot(values, wo_ref[...],
                  preferred_element_type=jnp.float32)
    o_ref[...] = out + bo_ref[0, :][None, :]


def kernel(x, padding_mask, Wqkv, bqkv, Wo, bo):
    w = Wqkv.reshape(_H, 3, _HD, _D)
    wq = w[:, 0].reshape(_D, _D).T.astype(jnp.bfloat16)
    wk = w[:, 1].reshape(_D, _D).T.astype(jnp.bfloat16)
    wv = w[:, 2].reshape(_D, _D).T.astype(jnp.bfloat16)
    b3 = bqkv.reshape(_H, 3, _HD)
    bq = b3[:, 0].reshape(1, _D)
    bk = b3[:, 1].reshape(1, _D)
    bv = b3[:, 2].reshape(1, _D)

    wo = Wo.T.astype(jnp.bfloat16)
    bo2 = bo.reshape(1, _D)
    mask2 = padding_mask.reshape(_S, 1).astype(jnp.float32)
    x2 = x.reshape(_S, _D)

    out = pl.pallas_call(
        _attn_body,
        grid=(_NBLK,),
        in_specs=[
            pl.BlockSpec((_S, _D), lambda i: (0, 0)),
            pl.BlockSpec((_D, _D), lambda i: (0, 0)),
            pl.BlockSpec((_D, _D), lambda i: (0, 0)),
            pl.BlockSpec((_D, _D), lambda i: (0, 0)),
            pl.BlockSpec((1, _D), lambda i: (0, 0)),
            pl.BlockSpec((1, _D), lambda i: (0, 0)),
            pl.BlockSpec((1, _D), lambda i: (0, 0)),
            pl.BlockSpec((_D, _D), lambda i: (0, 0)),
            pl.BlockSpec((1, _D), lambda i: (0, 0)),
            pl.BlockSpec((_S, 1), lambda i: (0, 0)),
        ],
        out_specs=pl.BlockSpec((_BQ, _D), lambda i: (i, 0)),
        out_shape=jax.ShapeDtypeStruct((_S, _D), jnp.float32),
    )(x2, wq, wk, wv, bq, bk, bv, wo, bo2, mask2)

    return out.reshape(_B, _S, _D)


# speedup vs baseline: 2.2744x; 1.0725x over previous
import math

import jax
import jax.numpy as jnp
from jax.experimental import pallas as pl
from jax.experimental.pallas import tpu as pltpu

_B, _S, _D = 1, 2048, 768
_H = 12
_HD = _D // _H
_D3 = 3 * _D
_WINDOW = 64
_HALF = _WINDOW // 2

_BQ = 256
_LK = _BQ + _WINDOW
_NBLK = _S // _BQ
_SCALE = 1.0 / math.sqrt(_HD)


def _attn_body(x_ref, w_ref, sv_ref, bv_ref, wo_ref, bo_ref, mask_ref,
               o_ref, wb_s, wob_s, qkv_s):
    i = pl.program_id(0)

    @pl.when(i == 0)
    def _cast_weights():
        wb_s[...] = w_ref[...].astype(jnp.bfloat16)
        wob_s[...] = wo_ref[...].astype(jnp.bfloat16)

    qs = pl.multiple_of(i * _BQ, _BQ)
    start = pl.multiple_of(
        jnp.minimum(jnp.maximum(qs - _HALF, 0), _S - _LK), _HALF)
    q_off = pl.multiple_of(qs - start, _HALF)

    xs = x_ref[pl.ds(start, _LK), :].astype(jnp.bfloat16)
    qkv = jax.lax.dot_general(
        xs, wb_s[...], (((1,), (1,)), ((), ())),
        preferred_element_type=jnp.float32)
    qkv_s[...] = (qkv * sv_ref[0, :][None, :]
                  + bv_ref[0, :][None, :]).astype(jnp.bfloat16)

    i_abs = qs + jax.lax.broadcasted_iota(jnp.int32, (_BQ, _LK), 0)
    j_abs = start + jax.lax.broadcasted_iota(jnp.int32, (_BQ, _LK), 1)
    band = (j_abs >= i_abs - _HALF) & (j_abs <= i_abs + _HALF)
    kpad = jnp.transpose(mask_ref[pl.ds(start, _LK), :])
    valid = band & (kpad != 0)
    addmask = jnp.where(valid, 0.0, -1e30).astype(jnp.float32)

    qpad = mask_ref[pl.ds(qs, _BQ), :]

    vals = []
    for h in range(_H):
        base = h * 3 * _HD
        qt = qkv_s[pl.ds(q_off, _BQ), base:base + _HD]
        kt = qkv_s[:, base + _HD:base + 2 * _HD]
        vt = qkv_s[:, base + 2 * _HD:base + 3 * _HD]
        logits = jax.lax.dot_general(
            qt, kt, (((1,), (1,)), ((), ())),
            preferred_element_type=jnp.float32)
        eb = jnp.exp(logits + addmask).astype(jnp.bfloat16)
        s = jnp.sum(eb, axis=1, keepdims=True,
                    dtype=jnp.float32)
        u = jax.lax.dot_general(
            eb, vt, (((1,), (0,)), ((), ())),
            preferred_element_type=jnp.float32)
        vals.append((u * (qpad * (1.0 / s))).astype(jnp.bfloat16))

    values = jnp.concatenate(vals, axis=1)

    out = jax.lax.dot_general(
        values, wob_s[...], (((1,), (1,)), ((), ())),
        preferred_element_type=jnp.float32)
    o_ref[...] = out + bo_ref[0, :][None, :]


def kernel(x, padding_mask, Wqkv, bqkv, Wo, bo):
    col = jnp.arange(_D3) % (3 * _HD)
    svec = jnp.where(col < _HD, jnp.float32(_SCALE),
                     jnp.float32(1.0)).reshape(1, _D3)
    bvec = (bqkv.astype(jnp.float32) * svec[0]).reshape(1, _D3)

    bo2 = bo.reshape(1, _D)
    mask2 = padding_mask.reshape(_S, 1).astype(jnp.float32)
    x2 = x.reshape(_S, _D)

    out = pl.pallas_call(
        _attn_body,
        grid=(_NBLK,),
        in_specs=[
            pl.BlockSpec((_S, _D), lambda i: (0, 0)),
            pl.BlockSpec((_D3, _D), lambda i: (0, 0)),
            pl.BlockSpec((1, _D3), lambda i: (0, 0)),
            pl.BlockSpec((1, _D3), lambda i: (0, 0)),
            pl.BlockSpec((_D, _D), lambda i: (0, 0)),
            pl.BlockSpec((1, _D), lambda i: (0, 0)),
            pl.BlockSpec((_S, 1), lambda i: (0, 0)),
        ],
        out_specs=pl.BlockSpec((_BQ, _D), lambda i: (i, 0)),
        out_shape=jax.ShapeDtypeStruct((_S, _D), jnp.float32),
        scratch_shapes=[
            pltpu.VMEM((_D3, _D), jnp.bfloat16),
            pltpu.VMEM((_D, _D), jnp.bfloat16),
            pltpu.VMEM((_LK, _D3), jnp.bfloat16),
        ],
    )(x2, Wqkv, svec, bvec, Wo, bo2, mask2)

    return out.reshape(_B, _S, _D)
